# baseline (speedup 1.0000x reference)
import functools

import jax
import jax.numpy as jnp
import numpy as np
from jax import lax
from jax.experimental import pallas as pl
from jax.experimental.pallas import tpu as pltpu
from jax.experimental.pallas import tpu_sc as plsc

_NH, _NL, _NP, _D = 8, 4, 4, 32
_SHAPES = [[200, 200], [100, 100], [50, 50], [25, 25]]
_NW = 32
_BQ = 2


_BM_A = 512
_NVB = 105
_NV_PAD = _NVB * _BM_A


def _rne_bf16_bits(x):
    u = lax.bitcast_convert_type(x, jnp.uint32)
    return (u + 0x7FFF + ((u >> 16) & 1)) >> 16


def _valproj_body(x_ref, wlo_ref, whi_ref, blo_ref, bhi_ref, o_ref):
    x = x_ref[...]
    lo = jnp.dot(x, wlo_ref[...], preferred_element_type=jnp.float32) + blo_ref[...]
    hi = jnp.dot(x, whi_ref[...], preferred_element_type=jnp.float32) + bhi_ref[...]
    packed = lax.bitcast_convert_type(
        _rne_bf16_bits(lo) | (_rne_bf16_bits(hi) << 16), jnp.float32)
    o_ref[...] = packed.reshape(_BM_A // 8, 8, 128)


def _valproj(vflat, W_lo, W_hi, b_lo, b_hi):
    m, c = vflat.shape
    tiles = (m + 7) // 8
    return pl.pallas_call(
        _valproj_body,
        grid=(pl.cdiv(m, _BM_A),),
        in_specs=[pl.BlockSpec((_BM_A, c), lambda i: (i, 0)),
                  pl.BlockSpec((c, 128), lambda i: (0, 0)),
                  pl.BlockSpec((c, 128), lambda i: (0, 0)),
                  pl.BlockSpec((1, 128), lambda i: (0, 0)),
                  pl.BlockSpec((1, 128), lambda i: (0, 0))],
        out_specs=pl.BlockSpec((_BM_A // 8, 8, 128), lambda i: (i, 0, 0)),
        out_shape=jax.ShapeDtypeStruct((tiles, 8, 128), jnp.float32),
    )(vflat, W_lo, W_hi, b_lo, b_hi)


def _sample_body(nq, q_ref, wox_ref, box_ref, woy_ref, boy_ref, wa_ref, ba_ref,
                 g_ref, pxb_ref, pyb_ref, wl_ref, hl_ref, iw2_ref, us2_ref,
                 hc_ref,
                 i00_ref, i10_ref, i01_ref, i11_ref,
                 w00_ref, w10_ref, w01_ref, w11_ref):
    q = q_ref[...]
    offx = jnp.dot(q, wox_ref[...], preferred_element_type=jnp.float32) + box_ref[...]
    offy = jnp.dot(q, woy_ref[...], preferred_element_type=jnp.float32) + boy_ref[...]
    e = jnp.exp(jnp.dot(q, wa_ref[...], preferred_element_type=jnp.float32) + ba_ref[...])
    aw = e / jnp.dot(e, g_ref[...], preferred_element_type=jnp.float32)
    wl = wl_ref[...]
    hl = hl_ref[...]
    px = pxb_ref[...] + offx
    py = pyb_ref[...] + offy
    x0 = jnp.floor(px)
    y0 = jnp.floor(py)
    fx = px - x0
    fy = py - y0
    wx0 = 1.0 - fx
    wy0 = 1.0 - fy
    vx0 = (x0 >= 0.0) & (x0 <= wl - 1.0)
    vx1 = (x0 + 1.0 >= 0.0) & (x0 + 1.0 <= wl - 1.0)
    vy0 = (y0 >= 0.0) & (y0 <= hl - 1.0)
    vy1 = (y0 + 1.0 >= 0.0) & (y0 + 1.0 <= hl - 1.0)
    xc0 = jnp.clip(x0, 0.0, wl - 1.0).astype(jnp.int32)
    xc1 = jnp.clip(x0 + 1.0, 0.0, wl - 1.0).astype(jnp.int32)
    yc0 = jnp.clip(y0, 0.0, hl - 1.0).astype(jnp.int32)
    yc1 = jnp.clip(y0 + 1.0, 0.0, hl - 1.0).astype(jnp.int32)
    iw2 = iw2_ref[...]
    rows_per_b = nq // q_ref.shape[0]
    b = pl.program_id(0) // rows_per_b
    uoff = us2_ref[...] + b
    hc = hc_ref[...]

    def pack(yc, xc):
        u = yc * iw2 + xc * 2 + uoff
        return ((u >> 3) << 6) + ((u & 7) << 3) + hc

    i00_ref[...] = pack(yc0, xc0)
    i10_ref[...] = pack(yc0, xc1)
    i01_ref[...] = pack(yc1, xc0)
    i11_ref[...] = pack(yc1, xc1)
    zero = jnp.zeros_like(aw)
    w00_ref[...] = jnp.where(vx0 & vy0, aw * wx0 * wy0, zero)
    w10_ref[...] = jnp.where(vx1 & vy0, aw * fx * wy0, zero)
    w01_ref[...] = jnp.where(vx0 & vy1, aw * wx0 * fy, zero)
    w11_ref[...] = jnp.where(vx1 & vy1, aw * fx * fy, zero)


def _sample(q, wox, box, woy, boy, wa, ba, g, pxb, pyb, wl, hl, iw2, us2, hc, nq):
    n, c = q.shape
    bm = 256
    row = lambda i: (i, 0)
    const = lambda i: (0, 0)
    io = [jax.ShapeDtypeStruct((n, 128), jnp.int32)] * 4 + \
         [jax.ShapeDtypeStruct((n, 128), jnp.float32)] * 4
    return pl.pallas_call(
        functools.partial(_sample_body, nq),
        grid=(n // bm,),
        in_specs=[pl.BlockSpec((bm, c), row),
                  pl.BlockSpec((c, 128), const), pl.BlockSpec((1, 128), const),
                  pl.BlockSpec((c, 128), const), pl.BlockSpec((1, 128), const),
                  pl.BlockSpec((c, 128), const), pl.BlockSpec((1, 128), const),
                  pl.BlockSpec((128, 128), const),
                  pl.BlockSpec((bm, 128), row), pl.BlockSpec((bm, 128), row),
                  pl.BlockSpec((1, 128), const), pl.BlockSpec((1, 128), const),
                  pl.BlockSpec((1, 128), const), pl.BlockSpec((1, 128), const),
                  pl.BlockSpec((1, 128), const)],
        out_specs=[pl.BlockSpec((bm, 128), row)] * 8,
        out_shape=io,
    )(q, wox, box, woy, boy, wa, ba, g, pxb, pyb, wl, hl, iw2, us2, hc)


def _sc_stage(bq0, idx_refs, w_refs, idxv, wv):
    for c in range(4):
        pltpu.sync_copy(idx_refs[c].at[pl.ds(bq0, _BQ)],
                        idxv.at[pl.ds(c * _BQ, _BQ)])
        pltpu.sync_copy(w_refs[c].at[pl.ds(bq0, _BQ)],
                        wv.at[pl.ds(c * _BQ, _BQ)])


def _sc_fire(table, idxv, gbuf, sem):
    return [pltpu.async_copy(table.at[idxv.at[t]],
                             gbuf.at[pl.ds(t * 128, 128)], sem)
            for t in range(4 * _BQ)]


def _sc_compute(gbuf, wv, obuf):
    nrow = 8 * _BQ

    def row_body(r, carry):
        bq = r >> 3
        h = r & 7
        acc0 = jnp.zeros((16,), jnp.float32)
        acc1 = jnp.zeros((16,), jnp.float32)
        for c in range(4):
            wq = wv[c * _BQ + bq, pl.ds(h * 16, 16)]
            gb = (c * _BQ + bq) * 128 + h * 16
            for s in range(16):
                wvec = jnp.take_along_axis(
                    wq, jnp.full((16,), s, jnp.int32), axis=0)
                pw = gbuf[gb + s, :]
                a, bv = plsc.unpack(plsc.bitcast(pw, jnp.bfloat16),
                                    format=plsc.PackFormat.INTERLEAVED)
                acc0 = acc0 + wvec * a
                acc1 = acc1 + wvec * bv
        orow = bq * 2 + (h >> 2)
        ocol = (h & 3) * 32
        obuf[orow, pl.ds(ocol, 16)] = acc0
        obuf[orow, pl.ds(ocol + 16, 16)] = acc1
        return carry

    lax.fori_loop(0, nrow, row_body, 0)


def _sc_gather_fn(nbq, table_hbm, i00, i10, i01, i11, w00, w10, w01, w11,
                  out_hbm, idxv0, idxv1, wv0, wv1, gbuf0, gbuf1,
                  obuf0, obuf1, sem0, sem1):
    idx_refs = (i00, i10, i01, i11)
    w_refs = (w00, w10, w01, w11)
    per_w = nbq // _NW
    nchunk = per_w // _BQ
    wid = lax.axis_index("s") * 2 + lax.axis_index("c")
    base_bq = wid * per_w

    def fire(idxv, gbuf, sem):
        _sc_fire(table_hbm, idxv, gbuf, sem)

    def drain(idxv, gbuf, sem):
        for t in range(4 * _BQ):
            pltpu.make_async_copy(table_hbm.at[idxv.at[t]],
                                  gbuf.at[pl.ds(t * 128, 128)], sem).wait()

    _sc_stage(base_bq, idx_refs, w_refs, idxv0, wv0)
    fire(idxv0, gbuf0, sem0)

    nhalf = nchunk // 2

    def body(k, carry):
        bq_a = base_bq + (2 * k) * _BQ
        bq_b = bq_a + _BQ
        _sc_stage(bq_b, idx_refs, w_refs, idxv1, wv1)
        fire(idxv1, gbuf1, sem1)
        drain(idxv0, gbuf0, sem0)
        _sc_compute(gbuf0, wv0, obuf0)
        pltpu.sync_copy(obuf0, out_hbm.at[pl.ds(bq_a * 2, 2 * _BQ)])

        @pl.when(k < nhalf - 1)
        def _():
            bq_c = bq_b + _BQ
            _sc_stage(bq_c, idx_refs, w_refs, idxv0, wv0)
            fire(idxv0, gbuf0, sem0)

        drain(idxv1, gbuf1, sem1)
        _sc_compute(gbuf1, wv1, obuf1)
        pltpu.sync_copy(obuf1, out_hbm.at[pl.ds(bq_b * 2, 2 * _BQ)])
        return carry

    lax.fori_loop(0, nhalf, body, 0)


def _sc_gather(table, i00, i10, i01, i11, w00, w10, w01, w11, nbq):
    mesh = plsc.VectorSubcoreMesh(core_axis_name="c", subcore_axis_name="s")
    fn = pl.kernel(
        functools.partial(_sc_gather_fn, nbq),
        mesh=mesh,
        compiler_params=pltpu.CompilerParams(needs_layout_passes=False,
                                             use_tc_tiling_on_sc=False),
        out_type=jax.ShapeDtypeStruct((nbq * 2, 128), jnp.float32),
        scratch_types=[
            pltpu.VMEM((4 * _BQ, 128), jnp.int32),
            pltpu.VMEM((4 * _BQ, 128), jnp.int32),
            pltpu.VMEM((4 * _BQ, 128), jnp.float32),
            pltpu.VMEM((4 * _BQ, 128), jnp.float32),
            pltpu.VMEM((4 * _BQ * 128, 16), jnp.float32),
            pltpu.VMEM((4 * _BQ * 128, 16), jnp.float32),
            pltpu.VMEM((2 * _BQ, 128), jnp.float32),
            pltpu.VMEM((2 * _BQ, 128), jnp.float32),
            pltpu.SemaphoreType.DMA,
            pltpu.SemaphoreType.DMA,
        ],
    )
    return fn(table, i00, i10, i01, i11, w00, w10, w01, w11)


def _outproj_body(x_ref, w_ref, b_ref, g_ref, bb_ref, id_ref, y_ref):
    x3 = x_ref[...].reshape(256, 2, 128)
    x = jnp.concatenate([x3[:, 0, :], x3[:, 1, :]], axis=1)
    o = jnp.dot(x, w_ref[...], preferred_element_type=jnp.float32) + b_ref[...]
    mu = jnp.mean(o, axis=-1, keepdims=True)
    var = jnp.mean((o - mu) ** 2, axis=-1, keepdims=True)
    o = (o - mu) * lax.rsqrt(var + 1e-5) * g_ref[...] + bb_ref[...]
    y_ref[...] = jnp.maximum(o, 0.0) + id_ref[...]


def _outproj(x2, W_out, b_out, ln_g, ln_b, ident):
    n, c = ident.shape
    bm = 256
    row = lambda i: (i, 0)
    const = lambda i: (0, 0)
    return pl.pallas_call(
        _outproj_body,
        grid=(n // bm,),
        in_specs=[pl.BlockSpec((2 * bm, 128), row),
                  pl.BlockSpec((c, c), const), pl.BlockSpec((1, c), const),
                  pl.BlockSpec((1, c), const), pl.BlockSpec((1, c), const),
                  pl.BlockSpec((bm, c), row)],
        out_specs=pl.BlockSpec((bm, c), row),
        out_shape=jax.ShapeDtypeStruct((n, c), jnp.float32),
    )(x2, W_out, b_out.reshape(1, c), ln_g.reshape(1, c), ln_b.reshape(1, c),
      ident)


def kernel(query, value, spatial_shapes, level_start_index, reference_trajs,
           det_centers, W_off, b_off, W_attn, b_attn, W_val, b_val, W_out,
           b_out, ln_g, ln_b):
    bs, A, M, C = query.shape
    nq = A * M
    NQ = bs * nq
    nv = value.shape[0]

    wcols = ((np.arange(128) // 16) * 32 + np.arange(128) % 16)
    W_lo = W_val[:, wcols]
    W_hi = W_val[:, wcols + 16]
    b_lo = b_val[wcols].reshape(1, 128)
    b_hi = b_val[wcols + 16].reshape(1, 128)
    t3 = _valproj(value.reshape(nv * bs, C), W_lo, W_hi, b_lo, b_hi)
    table = t3.reshape(t3.shape[0] * 64, 16)

    rt = reference_trajs[:, :, :, -1, :, :] + det_centers[:, :, None, None, :]
    rx = (rt[..., 0] + 51.2) / 102.4
    ry = (rt[..., 1] + 51.2) / 102.4
    wl_np = np.array([s[1] for s in _SHAPES], np.float32)
    hl_np = np.array([s[0] for s in _SHAPES], np.float32)
    pxb = (rx * wl_np - 0.5).reshape(bs, nq, 1, _NL, 1)
    pyb = (ry * hl_np - 0.5).reshape(bs, nq, 1, _NL, 1)
    pxb = jnp.broadcast_to(pxb, (bs, nq, _NH, _NL, _NP)).reshape(NQ, 128)
    pyb = jnp.broadcast_to(pyb, (bs, nq, _NH, _NL, _NP)).reshape(NQ, 128)

    l_of_col = np.tile(np.repeat(np.arange(_NL), _NP), _NH)
    h_of_col = np.repeat(np.arange(_NH), _NL * _NP)
    wl_col = jnp.asarray(wl_np[l_of_col].reshape(1, 128))
    hl_col = jnp.asarray(hl_np[l_of_col].reshape(1, 128))
    iw2_col = jnp.asarray((wl_np[l_of_col].astype(np.int64) * 2)
                          .astype(np.int32).reshape(1, 128))
    starts = np.concatenate([[0], np.cumsum([h * w for h, w in _SHAPES])[:-1]])
    us2 = jnp.asarray((starts[l_of_col] * 2).astype(np.int32).reshape(1, 128))
    hc = jnp.asarray(h_of_col.astype(np.int32).reshape(1, 128))

    wox = W_off.reshape(C, 128, 2)[:, :, 0]
    woy = W_off.reshape(C, 128, 2)[:, :, 1]
    box = b_off.reshape(128, 2)[:, 0].reshape(1, 128)
    boy = b_off.reshape(128, 2)[:, 1].reshape(1, 128)
    gmat = jnp.asarray(np.kron(np.eye(_NH, dtype=np.float32),
                               np.ones((16, 16), np.float32)))

    i00, i10, i01, i11, w00, w10, w01, w11 = _sample(
        query.reshape(NQ, C), wox, box, woy, boy, W_attn,
        b_attn.reshape(1, 128), gmat, pxb, pyb, wl_col, hl_col, iw2_col,
        us2, hc, nq)

    sc_out = _sc_gather(table, i00, i10, i01, i11,
                        w00, w10, w01, w11, NQ)

    out = _outproj(sc_out, W_out, b_out, ln_g, ln_b, query.reshape(NQ, C))
    return out.reshape(bs, A, M, C)


# speedup vs baseline: 1.1622x; 1.1622x over previous
import functools

import jax
import jax.numpy as jnp
import numpy as np
from jax import lax
from jax.experimental import pallas as pl
from jax.experimental.pallas import tpu as pltpu
from jax.experimental.pallas import tpu_sc as plsc

_NH, _NL, _NP, _D = 8, 4, 4, 32
_SHAPES = [[200, 200], [100, 100], [50, 50], [25, 25]]
_NW = 32
_BQ = 4


_BM_A = 512
_NVB = 105
_NV_PAD = _NVB * _BM_A


def _rne_bf16_bits(x):
    u = lax.bitcast_convert_type(x, jnp.uint32)
    return (u + 0x7FFF + ((u >> 16) & 1)) >> 16


def _valproj_body(x_ref, wlo_ref, whi_ref, blo_ref, bhi_ref, o_ref):
    x = x_ref[...]
    lo = jnp.dot(x, wlo_ref[...], preferred_element_type=jnp.float32) + blo_ref[...]
    hi = jnp.dot(x, whi_ref[...], preferred_element_type=jnp.float32) + bhi_ref[...]
    packed = lax.bitcast_convert_type(
        _rne_bf16_bits(lo) | (_rne_bf16_bits(hi) << 16), jnp.float32)
    o_ref[...] = packed.reshape(_BM_A // 8, 8, 128)


def _valproj(vflat, W_lo, W_hi, b_lo, b_hi):
    m, c = vflat.shape
    tiles = (m + 7) // 8
    return pl.pallas_call(
        _valproj_body,
        grid=(pl.cdiv(m, _BM_A),),
        in_specs=[pl.BlockSpec((_BM_A, c), lambda i: (i, 0)),
                  pl.BlockSpec((c, 128), lambda i: (0, 0)),
                  pl.BlockSpec((c, 128), lambda i: (0, 0)),
                  pl.BlockSpec((1, 128), lambda i: (0, 0)),
                  pl.BlockSpec((1, 128), lambda i: (0, 0))],
        out_specs=pl.BlockSpec((_BM_A // 8, 8, 128), lambda i: (i, 0, 0)),
        out_shape=jax.ShapeDtypeStruct((tiles, 8, 128), jnp.float32),
    )(vflat, W_lo, W_hi, b_lo, b_hi)


def _sample_body(nq, q_ref, wox_ref, box_ref, woy_ref, boy_ref, wa_ref, ba_ref,
                 g_ref, pxb_ref, pyb_ref, wl_ref, hl_ref, iw2_ref, us2_ref,
                 hc_ref,
                 i00_ref, i10_ref, i01_ref, i11_ref,
                 w00_ref, w10_ref, w01_ref, w11_ref):
    q = q_ref[...]
    offx = jnp.dot(q, wox_ref[...], preferred_element_type=jnp.float32) + box_ref[...]
    offy = jnp.dot(q, woy_ref[...], preferred_element_type=jnp.float32) + boy_ref[...]
    e = jnp.exp(jnp.dot(q, wa_ref[...], preferred_element_type=jnp.float32) + ba_ref[...])
    aw = e / jnp.dot(e, g_ref[...], preferred_element_type=jnp.float32)
    wl = wl_ref[...]
    hl = hl_ref[...]
    px = pxb_ref[...] + offx
    py = pyb_ref[...] + offy
    x0 = jnp.floor(px)
    y0 = jnp.floor(py)
    fx = px - x0
    fy = py - y0
    wx0 = 1.0 - fx
    wy0 = 1.0 - fy
    vx0 = (x0 >= 0.0) & (x0 <= wl - 1.0)
    vx1 = (x0 + 1.0 >= 0.0) & (x0 + 1.0 <= wl - 1.0)
    vy0 = (y0 >= 0.0) & (y0 <= hl - 1.0)
    vy1 = (y0 + 1.0 >= 0.0) & (y0 + 1.0 <= hl - 1.0)
    xc0 = jnp.clip(x0, 0.0, wl - 1.0).astype(jnp.int32)
    xc1 = jnp.clip(x0 + 1.0, 0.0, wl - 1.0).astype(jnp.int32)
    yc0 = jnp.clip(y0, 0.0, hl - 1.0).astype(jnp.int32)
    yc1 = jnp.clip(y0 + 1.0, 0.0, hl - 1.0).astype(jnp.int32)
    iw2 = iw2_ref[...]
    rows_per_b = nq // q_ref.shape[0]
    b = pl.program_id(0) // rows_per_b
    uoff = us2_ref[...] + b
    hc = hc_ref[...]

    def pack(yc, xc):
        u = yc * iw2 + xc * 2 + uoff
        return ((u >> 3) << 6) + ((u & 7) << 3) + hc

    i00_ref[...] = pack(yc0, xc0)
    i10_ref[...] = pack(yc0, xc1)
    i01_ref[...] = pack(yc1, xc0)
    i11_ref[...] = pack(yc1, xc1)
    zero = jnp.zeros_like(aw)
    w00_ref[...] = jnp.where(vx0 & vy0, aw * wx0 * wy0, zero)
    w10_ref[...] = jnp.where(vx1 & vy0, aw * fx * wy0, zero)
    w01_ref[...] = jnp.where(vx0 & vy1, aw * wx0 * fy, zero)
    w11_ref[...] = jnp.where(vx1 & vy1, aw * fx * fy, zero)


def _sample(q, wox, box, woy, boy, wa, ba, g, pxb, pyb, wl, hl, iw2, us2, hc, nq):
    n, c = q.shape
    bm = 256
    row = lambda i: (i, 0)
    const = lambda i: (0, 0)
    io = [jax.ShapeDtypeStruct((n, 128), jnp.int32)] * 4 + \
         [jax.ShapeDtypeStruct((n, 128), jnp.float32)] * 4
    return pl.pallas_call(
        functools.partial(_sample_body, nq),
        grid=(n // bm,),
        in_specs=[pl.BlockSpec((bm, c), row),
                  pl.BlockSpec((c, 128), const), pl.BlockSpec((1, 128), const),
                  pl.BlockSpec((c, 128), const), pl.BlockSpec((1, 128), const),
                  pl.BlockSpec((c, 128), const), pl.BlockSpec((1, 128), const),
                  pl.BlockSpec((128, 128), const),
                  pl.BlockSpec((bm, 128), row), pl.BlockSpec((bm, 128), row),
                  pl.BlockSpec((1, 128), const), pl.BlockSpec((1, 128), const),
                  pl.BlockSpec((1, 128), const), pl.BlockSpec((1, 128), const),
                  pl.BlockSpec((1, 128), const)],
        out_specs=[pl.BlockSpec((bm, 128), row)] * 8,
        out_shape=io,
    )(q, wox, box, woy, boy, wa, ba, g, pxb, pyb, wl, hl, iw2, us2, hc)


def _sc_stage(bq0, idx_refs, w_refs, idxv, wv):
    for c in range(4):
        pltpu.sync_copy(idx_refs[c].at[pl.ds(bq0, _BQ)],
                        idxv.at[pl.ds(c * _BQ, _BQ)])
        pltpu.sync_copy(w_refs[c].at[pl.ds(bq0, _BQ)],
                        wv.at[pl.ds(c * _BQ, _BQ)])


def _sc_fire(table, idxv, gbuf, sem):
    return [pltpu.async_copy(table.at[idxv.at[t]],
                             gbuf.at[pl.ds(t * 128, 128)], sem)
            for t in range(4 * _BQ)]


def _sc_compute(gbuf, wv, obuf):
    nrow = 8 * _BQ

    def row_body(r, carry):
        bq = r >> 3
        h = r & 7
        zero = jnp.zeros((16,), jnp.float32)
        lo_accs = []
        hi_accs = []
        for c in range(4):
            wq = wv[c * _BQ + bq, pl.ds(h * 16, 16)]
            gb = (c * _BQ + bq) * 128 + h * 16
            lo_c = zero
            hi_c = zero
            for s in range(16):
                wvec = jnp.take_along_axis(
                    wq, jnp.full((16,), s, jnp.int32), axis=0)
                pw = gbuf[gb + s, :]
                a, bv = plsc.unpack(plsc.bitcast(pw, jnp.bfloat16),
                                    format=plsc.PackFormat.INTERLEAVED)
                lo_c = lo_c + wvec * a
                hi_c = hi_c + wvec * bv
            lo_accs.append(lo_c)
            hi_accs.append(hi_c)
        acc0 = (lo_accs[0] + lo_accs[1]) + (lo_accs[2] + lo_accs[3])
        acc1 = (hi_accs[0] + hi_accs[1]) + (hi_accs[2] + hi_accs[3])
        orow = bq * 2 + (h >> 2)
        ocol = (h & 3) * 32
        obuf[orow, pl.ds(ocol, 16)] = acc0
        obuf[orow, pl.ds(ocol + 16, 16)] = acc1
        return carry

    lax.fori_loop(0, nrow, row_body, 0)


def _sc_gather_fn(nbq, table_hbm, i00, i10, i01, i11, w00, w10, w01, w11,
                  out_hbm, idxv0, idxv1, wv0, wv1, gbuf0, gbuf1,
                  obuf0, obuf1, sem0, sem1):
    idx_refs = (i00, i10, i01, i11)
    w_refs = (w00, w10, w01, w11)
    per_w = nbq // _NW
    nchunk = per_w // _BQ
    wid = lax.axis_index("s") * 2 + lax.axis_index("c")
    base_bq = wid * per_w

    def fire(idxv, gbuf, sem):
        _sc_fire(table_hbm, idxv, gbuf, sem)

    def drain(idxv, gbuf, sem):
        for t in range(4 * _BQ):
            pltpu.make_async_copy(table_hbm.at[idxv.at[t]],
                                  gbuf.at[pl.ds(t * 128, 128)], sem).wait()

    _sc_stage(base_bq, idx_refs, w_refs, idxv0, wv0)
    fire(idxv0, gbuf0, sem0)

    nhalf = nchunk // 2

    def body(k, carry):
        bq_a = base_bq + (2 * k) * _BQ
        bq_b = bq_a + _BQ
        _sc_stage(bq_b, idx_refs, w_refs, idxv1, wv1)
        fire(idxv1, gbuf1, sem1)
        drain(idxv0, gbuf0, sem0)
        _sc_compute(gbuf0, wv0, obuf0)
        pltpu.sync_copy(obuf0, out_hbm.at[pl.ds(bq_a * 2, 2 * _BQ)])

        @pl.when(k < nhalf - 1)
        def _():
            bq_c = bq_b + _BQ
            _sc_stage(bq_c, idx_refs, w_refs, idxv0, wv0)
            fire(idxv0, gbuf0, sem0)

        drain(idxv1, gbuf1, sem1)
        _sc_compute(gbuf1, wv1, obuf1)
        pltpu.sync_copy(obuf1, out_hbm.at[pl.ds(bq_b * 2, 2 * _BQ)])
        return carry

    lax.fori_loop(0, nhalf, body, 0)


def _sc_gather(table, i00, i10, i01, i11, w00, w10, w01, w11, nbq):
    mesh = plsc.VectorSubcoreMesh(core_axis_name="c", subcore_axis_name="s")
    fn = pl.kernel(
        functools.partial(_sc_gather_fn, nbq),
        mesh=mesh,
        compiler_params=pltpu.CompilerParams(needs_layout_passes=False,
                                             use_tc_tiling_on_sc=False),
        out_type=jax.ShapeDtypeStruct((nbq * 2, 128), jnp.float32),
        scratch_types=[
            pltpu.VMEM((4 * _BQ, 128), jnp.int32),
            pltpu.VMEM((4 * _BQ, 128), jnp.int32),
            pltpu.VMEM((4 * _BQ, 128), jnp.float32),
            pltpu.VMEM((4 * _BQ, 128), jnp.float32),
            pltpu.VMEM((4 * _BQ * 128, 16), jnp.float32),
            pltpu.VMEM((4 * _BQ * 128, 16), jnp.float32),
            pltpu.VMEM((2 * _BQ, 128), jnp.float32),
            pltpu.VMEM((2 * _BQ, 128), jnp.float32),
            pltpu.SemaphoreType.DMA,
            pltpu.SemaphoreType.DMA,
        ],
    )
    return fn(table, i00, i10, i01, i11, w00, w10, w01, w11)


def _outproj_body(x_ref, w_ref, b_ref, g_ref, bb_ref, id_ref, y_ref):
    x3 = x_ref[...].reshape(256, 2, 128)
    x = jnp.concatenate([x3[:, 0, :], x3[:, 1, :]], axis=1)
    o = jnp.dot(x, w_ref[...], preferred_element_type=jnp.float32) + b_ref[...]
    mu = jnp.mean(o, axis=-1, keepdims=True)
    var = jnp.mean((o - mu) ** 2, axis=-1, keepdims=True)
    o = (o - mu) * lax.rsqrt(var + 1e-5) * g_ref[...] + bb_ref[...]
    y_ref[...] = jnp.maximum(o, 0.0) + id_ref[...]


def _outproj(x2, W_out, b_out, ln_g, ln_b, ident):
    n, c = ident.shape
    bm = 256
    row = lambda i: (i, 0)
    const = lambda i: (0, 0)
    return pl.pallas_call(
        _outproj_body,
        grid=(n // bm,),
        in_specs=[pl.BlockSpec((2 * bm, 128), row),
                  pl.BlockSpec((c, c), const), pl.BlockSpec((1, c), const),
                  pl.BlockSpec((1, c), const), pl.BlockSpec((1, c), const),
                  pl.BlockSpec((bm, c), row)],
        out_specs=pl.BlockSpec((bm, c), row),
        out_shape=jax.ShapeDtypeStruct((n, c), jnp.float32),
    )(x2, W_out, b_out.reshape(1, c), ln_g.reshape(1, c), ln_b.reshape(1, c),
      ident)


def kernel(query, value, spatial_shapes, level_start_index, reference_trajs,
           det_centers, W_off, b_off, W_attn, b_attn, W_val, b_val, W_out,
           b_out, ln_g, ln_b):
    bs, A, M, C = query.shape
    nq = A * M
    NQ = bs * nq
    nv = value.shape[0]

    wcols = ((np.arange(128) // 16) * 32 + np.arange(128) % 16)
    W_lo = W_val[:, wcols]
    W_hi = W_val[:, wcols + 16]
    b_lo = b_val[wcols].reshape(1, 128)
    b_hi = b_val[wcols + 16].reshape(1, 128)
    t3 = _valproj(value.reshape(nv * bs, C), W_lo, W_hi, b_lo, b_hi)
    table = t3.reshape(t3.shape[0] * 64, 16)

    rt = reference_trajs[:, :, :, -1, :, :] + det_centers[:, :, None, None, :]
    rx = (rt[..., 0] + 51.2) / 102.4
    ry = (rt[..., 1] + 51.2) / 102.4
    wl_np = np.array([s[1] for s in _SHAPES], np.float32)
    hl_np = np.array([s[0] for s in _SHAPES], np.float32)
    pxb = (rx * wl_np - 0.5).reshape(bs, nq, 1, _NL, 1)
    pyb = (ry * hl_np - 0.5).reshape(bs, nq, 1, _NL, 1)
    pxb = jnp.broadcast_to(pxb, (bs, nq, _NH, _NL, _NP)).reshape(NQ, 128)
    pyb = jnp.broadcast_to(pyb, (bs, nq, _NH, _NL, _NP)).reshape(NQ, 128)

    l_of_col = np.tile(np.repeat(np.arange(_NL), _NP), _NH)
    h_of_col = np.repeat(np.arange(_NH), _NL * _NP)
    wl_col = jnp.asarray(wl_np[l_of_col].reshape(1, 128))
    hl_col = jnp.asarray(hl_np[l_of_col].reshape(1, 128))
    iw2_col = jnp.asarray((wl_np[l_of_col].astype(np.int64) * 2)
                          .astype(np.int32).reshape(1, 128))
    starts = np.concatenate([[0], np.cumsum([h * w for h, w in _SHAPES])[:-1]])
    us2 = jnp.asarray((starts[l_of_col] * 2).astype(np.int32).reshape(1, 128))
    hc = jnp.asarray(h_of_col.astype(np.int32).reshape(1, 128))

    wox = W_off.reshape(C, 128, 2)[:, :, 0]
    woy = W_off.reshape(C, 128, 2)[:, :, 1]
    box = b_off.reshape(128, 2)[:, 0].reshape(1, 128)
    boy = b_off.reshape(128, 2)[:, 1].reshape(1, 128)
    gmat = jnp.asarray(np.kron(np.eye(_NH, dtype=np.float32),
                               np.ones((16, 16), np.float32)))

    i00, i10, i01, i11, w00, w10, w01, w11 = _sample(
        query.reshape(NQ, C), wox, box, woy, boy, W_attn,
        b_attn.reshape(1, 128), gmat, pxb, pyb, wl_col, hl_col, iw2_col,
        us2, hc, nq)

    sc_out = _sc_gather(table, i00, i10, i01, i11,
                        w00, w10, w01, w11, NQ)

    out = _outproj(sc_out, W_out, b_out, ln_g, ln_b, query.reshape(NQ, C))
    return out.reshape(bs, A, M, C)


# speedup vs baseline: 1.2957x; 1.1149x over previous
import functools

import jax
import jax.numpy as jnp
import numpy as np
from jax import lax
from jax.experimental import pallas as pl
from jax.experimental.pallas import tpu as pltpu
from jax.experimental.pallas import tpu_sc as plsc

_NH, _NL, _NP, _D = 8, 4, 4, 32
_SHAPES = [[200, 200], [100, 100], [50, 50], [25, 25]]
_NW = 32
_BQ = 4


_BM_A = 1024
_NVB = 105
_NV_PAD = _NVB * 512


def _rne_bf16_bits(x):
    u = lax.bitcast_convert_type(x, jnp.uint32)
    return (u + 0x7FFF + ((u >> 16) & 1)) >> 16


def _valproj_body(x_ref, wlo_ref, whi_ref, blo_ref, bhi_ref, o_ref):
    x = x_ref[...]
    lo = jnp.dot(x, wlo_ref[...], preferred_element_type=jnp.float32) + blo_ref[...]
    hi = jnp.dot(x, whi_ref[...], preferred_element_type=jnp.float32) + bhi_ref[...]
    packed = lax.bitcast_convert_type(
        _rne_bf16_bits(lo) | (_rne_bf16_bits(hi) << 16), jnp.float32)
    o_ref[...] = packed.reshape(_BM_A // 8, 8, 128)


def _valproj(vflat, W_lo, W_hi, b_lo, b_hi):
    m, c = vflat.shape
    tiles = (m + 7) // 8
    return pl.pallas_call(
        _valproj_body,
        grid=(pl.cdiv(m, _BM_A),),
        in_specs=[pl.BlockSpec((_BM_A, c), lambda i: (i, 0)),
                  pl.BlockSpec((c, 128), lambda i: (0, 0)),
                  pl.BlockSpec((c, 128), lambda i: (0, 0)),
                  pl.BlockSpec((1, 128), lambda i: (0, 0)),
                  pl.BlockSpec((1, 128), lambda i: (0, 0))],
        out_specs=pl.BlockSpec((_BM_A // 8, 8, 128), lambda i: (i, 0, 0)),
        out_shape=jax.ShapeDtypeStruct((tiles, 8, 128), jnp.float32),
    )(vflat, W_lo, W_hi, b_lo, b_hi)


def _sample_body(nq, q_ref, wox_ref, box_ref, woy_ref, boy_ref, wa_ref, ba_ref,
                 g_ref, pxb_ref, pyb_ref, wl_ref, hl_ref, iw2_ref, us2_ref,
                 hc_ref,
                 i00_ref, i10_ref, i01_ref, i11_ref,
                 w00_ref, w10_ref, w01_ref, w11_ref):
    q = q_ref[...]
    offx = jnp.dot(q, wox_ref[...], preferred_element_type=jnp.float32) + box_ref[...]
    offy = jnp.dot(q, woy_ref[...], preferred_element_type=jnp.float32) + boy_ref[...]
    e = jnp.exp(jnp.dot(q, wa_ref[...], preferred_element_type=jnp.float32) + ba_ref[...])
    aw = e / jnp.dot(e, g_ref[...], preferred_element_type=jnp.float32)
    wl = wl_ref[...]
    hl = hl_ref[...]
    px = pxb_ref[...] + offx
    py = pyb_ref[...] + offy
    x0 = jnp.floor(px)
    y0 = jnp.floor(py)
    fx = px - x0
    fy = py - y0
    wx0 = 1.0 - fx
    wy0 = 1.0 - fy
    vx0 = (x0 >= 0.0) & (x0 <= wl - 1.0)
    vx1 = (x0 + 1.0 >= 0.0) & (x0 + 1.0 <= wl - 1.0)
    vy0 = (y0 >= 0.0) & (y0 <= hl - 1.0)
    vy1 = (y0 + 1.0 >= 0.0) & (y0 + 1.0 <= hl - 1.0)
    xc0 = jnp.clip(x0, 0.0, wl - 1.0).astype(jnp.int32)
    xc1 = jnp.clip(x0 + 1.0, 0.0, wl - 1.0).astype(jnp.int32)
    yc0 = jnp.clip(y0, 0.0, hl - 1.0).astype(jnp.int32)
    yc1 = jnp.clip(y0 + 1.0, 0.0, hl - 1.0).astype(jnp.int32)
    iw2 = iw2_ref[...]
    rows_per_b = nq // q_ref.shape[0]
    b = pl.program_id(0) // rows_per_b
    uoff = us2_ref[...] + b
    hc = hc_ref[...]

    def pack(yc, xc):
        u = yc * iw2 + xc * 2 + uoff
        return ((u >> 3) << 6) + ((u & 7) << 3) + hc

    i00_ref[...] = pack(yc0, xc0)
    i10_ref[...] = pack(yc0, xc1)
    i01_ref[...] = pack(yc1, xc0)
    i11_ref[...] = pack(yc1, xc1)
    zero = jnp.zeros_like(aw)
    w00_ref[...] = jnp.where(vx0 & vy0, aw * wx0 * wy0, zero)
    w10_ref[...] = jnp.where(vx1 & vy0, aw * fx * wy0, zero)
    w01_ref[...] = jnp.where(vx0 & vy1, aw * wx0 * fy, zero)
    w11_ref[...] = jnp.where(vx1 & vy1, aw * fx * fy, zero)


def _sample(q, wox, box, woy, boy, wa, ba, g, pxb, pyb, wl, hl, iw2, us2, hc, nq):
    n, c = q.shape
    bm = 256
    row = lambda i: (i, 0)
    const = lambda i: (0, 0)
    io = [jax.ShapeDtypeStruct((n, 128), jnp.int32)] * 4 + \
         [jax.ShapeDtypeStruct((n, 128), jnp.float32)] * 4
    return pl.pallas_call(
        functools.partial(_sample_body, nq),
        grid=(n // bm,),
        in_specs=[pl.BlockSpec((bm, c), row),
                  pl.BlockSpec((c, 128), const), pl.BlockSpec((1, 128), const),
                  pl.BlockSpec((c, 128), const), pl.BlockSpec((1, 128), const),
                  pl.BlockSpec((c, 128), const), pl.BlockSpec((1, 128), const),
                  pl.BlockSpec((128, 128), const),
                  pl.BlockSpec((bm, 128), row), pl.BlockSpec((bm, 128), row),
                  pl.BlockSpec((1, 128), const), pl.BlockSpec((1, 128), const),
                  pl.BlockSpec((1, 128), const), pl.BlockSpec((1, 128), const),
                  pl.BlockSpec((1, 128), const)],
        out_specs=[pl.BlockSpec((bm, 128), row)] * 8,
        out_shape=io,
    )(q, wox, box, woy, boy, wa, ba, g, pxb, pyb, wl, hl, iw2, us2, hc)


def _sc_stage(bq0, idx_refs, w_refs, idxv, wv):
    for c in range(4):
        pltpu.sync_copy(idx_refs[c].at[pl.ds(bq0, _BQ)],
                        idxv.at[pl.ds(c * _BQ, _BQ)])
        pltpu.sync_copy(w_refs[c].at[pl.ds(bq0, _BQ)],
                        wv.at[pl.ds(c * _BQ, _BQ)])


def _sc_fire(table, idxv, gbuf, sem):
    return [pltpu.async_copy(table.at[idxv.at[t]],
                             gbuf.at[pl.ds(t * 128, 128)], sem)
            for t in range(4 * _BQ)]


def _sc_compute(gbuf, wv, obuf):
    nrow = 8 * _BQ

    def row_body(r, carry):
        bq = r >> 3
        h = r & 7
        zero = jnp.zeros((16,), jnp.float32)
        lo_accs = []
        hi_accs = []
        for c in range(4):
            wq = wv[c * _BQ + bq, pl.ds(h * 16, 16)]
            gb = (c * _BQ + bq) * 128 + h * 16
            lo_c = [zero, zero]
            hi_c = [zero, zero]
            for s in range(16):
                wvec = jnp.take_along_axis(
                    wq, jnp.full((16,), s, jnp.int32), axis=0)
                pw = gbuf[gb + s, :]
                a, bv = plsc.unpack(plsc.bitcast(pw, jnp.bfloat16),
                                    format=plsc.PackFormat.INTERLEAVED)
                lo_c[s & 1] = lo_c[s & 1] + wvec * a
                hi_c[s & 1] = hi_c[s & 1] + wvec * bv
            lo_accs.append(lo_c[0] + lo_c[1])
            hi_accs.append(hi_c[0] + hi_c[1])
        acc0 = (lo_accs[0] + lo_accs[1]) + (lo_accs[2] + lo_accs[3])
        acc1 = (hi_accs[0] + hi_accs[1]) + (hi_accs[2] + hi_accs[3])
        orow = bq * 2 + (h >> 2)
        ocol = (h & 3) * 32
        obuf[orow, pl.ds(ocol, 16)] = acc0
        obuf[orow, pl.ds(ocol + 16, 16)] = acc1
        return carry

    lax.fori_loop(0, nrow, row_body, 0)


def _sc_gather_fn(nbq, table_hbm, i00, i10, i01, i11, w00, w10, w01, w11,
                  out_hbm, idxv0, idxv1, wv0, wv1, gbuf0, gbuf1,
                  obuf0, obuf1, sem0, sem1):
    idx_refs = (i00, i10, i01, i11)
    w_refs = (w00, w10, w01, w11)
    per_w = nbq // _NW
    nchunk = per_w // _BQ
    wid = lax.axis_index("s") * 2 + lax.axis_index("c")
    base_bq = wid * per_w

    def fire(idxv, gbuf, sem):
        _sc_fire(table_hbm, idxv, gbuf, sem)

    def drain(idxv, gbuf, sem):
        for t in range(4 * _BQ):
            pltpu.make_async_copy(table_hbm.at[idxv.at[t]],
                                  gbuf.at[pl.ds(t * 128, 128)], sem).wait()

    _sc_stage(base_bq, idx_refs, w_refs, idxv0, wv0)
    fire(idxv0, gbuf0, sem0)

    nhalf = nchunk // 2

    def body(k, carry):
        bq_a = base_bq + (2 * k) * _BQ
        bq_b = bq_a + _BQ
        _sc_stage(bq_b, idx_refs, w_refs, idxv1, wv1)
        fire(idxv1, gbuf1, sem1)
        drain(idxv0, gbuf0, sem0)
        _sc_compute(gbuf0, wv0, obuf0)
        pltpu.sync_copy(obuf0, out_hbm.at[pl.ds(bq_a * 2, 2 * _BQ)])

        @pl.when(k < nhalf - 1)
        def _():
            bq_c = bq_b + _BQ
            _sc_stage(bq_c, idx_refs, w_refs, idxv0, wv0)
            fire(idxv0, gbuf0, sem0)

        drain(idxv1, gbuf1, sem1)
        _sc_compute(gbuf1, wv1, obuf1)
        pltpu.sync_copy(obuf1, out_hbm.at[pl.ds(bq_b * 2, 2 * _BQ)])
        return carry

    lax.fori_loop(0, nhalf, body, 0)


def _sc_gather(table, i00, i10, i01, i11, w00, w10, w01, w11, nbq):
    mesh = plsc.VectorSubcoreMesh(core_axis_name="c", subcore_axis_name="s")
    fn = pl.kernel(
        functools.partial(_sc_gather_fn, nbq),
        mesh=mesh,
        compiler_params=pltpu.CompilerParams(needs_layout_passes=False,
                                             use_tc_tiling_on_sc=False),
        out_type=jax.ShapeDtypeStruct((nbq * 2, 128), jnp.float32),
        scratch_types=[
            pltpu.VMEM((4 * _BQ, 128), jnp.int32),
            pltpu.VMEM((4 * _BQ, 128), jnp.int32),
            pltpu.VMEM((4 * _BQ, 128), jnp.float32),
            pltpu.VMEM((4 * _BQ, 128), jnp.float32),
            pltpu.VMEM((4 * _BQ * 128, 16), jnp.float32),
            pltpu.VMEM((4 * _BQ * 128, 16), jnp.float32),
            pltpu.VMEM((2 * _BQ, 128), jnp.float32),
            pltpu.VMEM((2 * _BQ, 128), jnp.float32),
            pltpu.SemaphoreType.DMA,
            pltpu.SemaphoreType.DMA,
        ],
    )
    return fn(table, i00, i10, i01, i11, w00, w10, w01, w11)


def _outproj_body(x_ref, w_ref, b_ref, g_ref, bb_ref, id_ref, y_ref):
    x3 = x_ref[...].reshape(256, 2, 128)
    x = jnp.concatenate([x3[:, 0, :], x3[:, 1, :]], axis=1)
    o = jnp.dot(x, w_ref[...], preferred_element_type=jnp.float32) + b_ref[...]
    mu = jnp.mean(o, axis=-1, keepdims=True)
    var = jnp.mean((o - mu) ** 2, axis=-1, keepdims=True)
    o = (o - mu) * lax.rsqrt(var + 1e-5) * g_ref[...] + bb_ref[...]
    y_ref[...] = jnp.maximum(o, 0.0) + id_ref[...]


def _outproj(x2, W_out, b_out, ln_g, ln_b, ident):
    n, c = ident.shape
    bm = 256
    row = lambda i: (i, 0)
    const = lambda i: (0, 0)
    return pl.pallas_call(
        _outproj_body,
        grid=(n // bm,),
        in_specs=[pl.BlockSpec((2 * bm, 128), row),
                  pl.BlockSpec((c, c), const), pl.BlockSpec((1, c), const),
                  pl.BlockSpec((1, c), const), pl.BlockSpec((1, c), const),
                  pl.BlockSpec((bm, c), row)],
        out_specs=pl.BlockSpec((bm, c), row),
        out_shape=jax.ShapeDtypeStruct((n, c), jnp.float32),
    )(x2, W_out, b_out.reshape(1, c), ln_g.reshape(1, c), ln_b.reshape(1, c),
      ident)


def kernel(query, value, spatial_shapes, level_start_index, reference_trajs,
           det_centers, W_off, b_off, W_attn, b_attn, W_val, b_val, W_out,
           b_out, ln_g, ln_b):
    bs, A, M, C = query.shape
    nq = A * M
    NQ = bs * nq
    nv = value.shape[0]

    wcols = ((np.arange(128) // 16) * 32 + np.arange(128) % 16)
    W_lo = W_val[:, wcols]
    W_hi = W_val[:, wcols + 16]
    b_lo = b_val[wcols].reshape(1, 128)
    b_hi = b_val[wcols + 16].reshape(1, 128)
    t3 = _valproj(value.reshape(nv * bs, C), W_lo, W_hi, b_lo, b_hi)
    table = t3.reshape(t3.shape[0] * 64, 16)

    rt = reference_trajs[:, :, :, -1, :, :] + det_centers[:, :, None, None, :]
    rx = (rt[..., 0] + 51.2) / 102.4
    ry = (rt[..., 1] + 51.2) / 102.4
    wl_np = np.array([s[1] for s in _SHAPES], np.float32)
    hl_np = np.array([s[0] for s in _SHAPES], np.float32)
    pxb = (rx * wl_np - 0.5).reshape(bs, nq, 1, _NL, 1)
    pyb = (ry * hl_np - 0.5).reshape(bs, nq, 1, _NL, 1)
    pxb = jnp.broadcast_to(pxb, (bs, nq, _NH, _NL, _NP)).reshape(NQ, 128)
    pyb = jnp.broadcast_to(pyb, (bs, nq, _NH, _NL, _NP)).reshape(NQ, 128)

    l_of_col = np.tile(np.repeat(np.arange(_NL), _NP), _NH)
    h_of_col = np.repeat(np.arange(_NH), _NL * _NP)
    wl_col = jnp.asarray(wl_np[l_of_col].reshape(1, 128))
    hl_col = jnp.asarray(hl_np[l_of_col].reshape(1, 128))
    iw2_col = jnp.asarray((wl_np[l_of_col].astype(np.int64) * 2)
                          .astype(np.int32).reshape(1, 128))
    starts = np.concatenate([[0], np.cumsum([h * w for h, w in _SHAPES])[:-1]])
    us2 = jnp.asarray((starts[l_of_col] * 2).astype(np.int32).reshape(1, 128))
    hc = jnp.asarray(h_of_col.astype(np.int32).reshape(1, 128))

    wox = W_off.reshape(C, 128, 2)[:, :, 0]
    woy = W_off.reshape(C, 128, 2)[:, :, 1]
    box = b_off.reshape(128, 2)[:, 0].reshape(1, 128)
    boy = b_off.reshape(128, 2)[:, 1].reshape(1, 128)
    gmat = jnp.asarray(np.kron(np.eye(_NH, dtype=np.float32),
                               np.ones((16, 16), np.float32)))

    i00, i10, i01, i11, w00, w10, w01, w11 = _sample(
        query.reshape(NQ, C), wox, box, woy, boy, W_attn,
        b_attn.reshape(1, 128), gmat, pxb, pyb, wl_col, hl_col, iw2_col,
        us2, hc, nq)

    sc_out = _sc_gather(table, i00, i10, i01, i11,
                        w00, w10, w01, w11, NQ)

    out = _outproj(sc_out, W_out, b_out, ln_g, ln_b, query.reshape(NQ, C))
    return out.reshape(bs, A, M, C)
